# Initial kernel scaffold; baseline (speedup 1.0000x reference)
import jax
import jax.numpy as jnp
from jax.experimental import pallas as pl


def kernel(x, token_table):
    raise NotImplementedError("write your pallas kernel here")


# speedup vs baseline: 1.7140x; 1.7140x over previous
import functools

import jax
import jax.numpy as jnp
import numpy as np
from jax import lax
from jax.experimental import pallas as pl
from jax.experimental.pallas import tpu as pltpu
from jax.experimental.pallas import tpu_sc as plsc

N_VOCAB = 100000
EMBED_DIM = 768
BATCH = 4
SEQ_LEN = 8192

NUM_WORKERS = 32
POS_PER_WORKER = SEQ_LEN // NUM_WORKERS
CHUNK = 64
N_CHUNKS = POS_PER_WORKER // CHUNK
LANES = 16
VECS_PER_ROW = EMBED_DIM // LANES


def _positional_encoding_np(max_len, d):
    pos = np.arange(max_len, dtype=np.float64)[:, None]
    i = np.arange(0, d, 2, dtype=np.float64)
    div = np.exp(-(np.log(10000.0) * i / d))
    ang = pos * div[None, :]
    pe = np.zeros((max_len, d), dtype=np.float64)
    pe[:, 0::2] = np.sin(ang)
    pe[:, 1::2] = np.cos(ang)
    return pe.astype(np.float32)


_PE = _positional_encoding_np(SEQ_LEN, EMBED_DIM)


def _sc_body(x_hbm, table_hbm, pe_hbm, out_hbm, idx_v, pe_v, rows_v, sem):
    wid = lax.axis_index("s") * 2 + lax.axis_index("c")
    pos0 = wid * POS_PER_WORKER

    def step(t, _):
        j = t // BATCH
        b = t % BATCH
        pos = pos0 + j * CHUNK

        @pl.when(b == 0)
        def _load_pe():
            pltpu.sync_copy(pe_hbm.at[pl.ds(pos, CHUNK)], pe_v)

        base = b * SEQ_LEN + pos
        pltpu.sync_copy(x_hbm.at[pl.ds(base, CHUNK)], idx_v)
        pltpu.async_copy(table_hbm.at[idx_v], rows_v, sem).wait()

        def add_row(r, _):
            for k in range(VECS_PER_ROW):
                sl = pl.ds(k * LANES, LANES)
                rows_v[r, sl] = rows_v[r, sl] + pe_v[r, sl]
            return _

        lax.fori_loop(0, CHUNK, add_row, None)
        pltpu.sync_copy(rows_v, out_hbm.at[pl.ds(base, CHUNK)])
        return _

    lax.fori_loop(0, N_CHUNKS * BATCH, step, None)


@functools.partial(jax.jit, static_argnames=())
def kernel(x, token_table):
    x_flat = x.reshape(-1).astype(jnp.int32)
    pe = jnp.asarray(_PE)

    mesh = plsc.VectorSubcoreMesh(core_axis_name="c", subcore_axis_name="s")
    run = pl.kernel(
        _sc_body,
        out_type=jax.ShapeDtypeStruct((BATCH * SEQ_LEN, EMBED_DIM), jnp.float32),
        mesh=mesh,
        scratch_types=[
            pltpu.VMEM((CHUNK,), jnp.int32),
            pltpu.VMEM((CHUNK, EMBED_DIM), jnp.float32),
            pltpu.VMEM((CHUNK, EMBED_DIM), jnp.float32),
            pltpu.SemaphoreType.DMA,
        ],
    )
    out = run(x_flat, token_table, pe)
    return out.reshape(BATCH, SEQ_LEN, EMBED_DIM)
